# baseline (speedup 1.0000x reference)
import functools

import jax
import jax.numpy as jnp
from jax import lax
from jax.experimental import pallas as pl
from jax.experimental.pallas import tpu as pltpu
from jax.experimental.pallas import tpu_sc as plsc

EMBED_NUM = 1000
EMBED_DIM = 64
BATCH = 16384
HIST = 200

_TCB = 1024
_SCB = BATCH - _TCB
_NC = 2
_NS = 16
_NW = _NC * _NS
_BPW = _SCB // _NW
_GE = 2
_NG = _BPW // _GE
_NI = _NG // 4
_SPLITS = ((0, 128), (128, 72))


def _sc_gather(idx, table):
    mesh = plsc.VectorSubcoreMesh(core_axis_name="c", subcore_axis_name="s")

    @functools.partial(
        pl.kernel,
        mesh=mesh,
        compiler_params=pltpu.CompilerParams(use_tc_tiling_on_sc=False),
        out_type=jax.ShapeDtypeStruct((_SCB, HIST, EMBED_DIM), jnp.float32),
        scratch_types=[
            pltpu.VMEM((4, _GE, HIST), jnp.int32),
            pltpu.VMEM((4, _GE, HIST, EMBED_DIM), jnp.float32),
            pltpu.SemaphoreType.DMA,
            pltpu.SemaphoreType.DMA,
            pltpu.SemaphoreType.DMA,
            pltpu.SemaphoreType.DMA,
            pltpu.SemaphoreType.DMA,
            pltpu.SemaphoreType.DMA,
            pltpu.SemaphoreType.DMA,
            pltpu.SemaphoreType.DMA,
            pltpu.SemaphoreType.DMA,
            pltpu.SemaphoreType.DMA,
            pltpu.SemaphoreType.DMA,
            pltpu.SemaphoreType.DMA,
        ],
    )
    def k(idx_hbm, table_hbm, out_hbm, idx_v, rows_v, *sems):
        sg = sems[0:4]
        sw = sems[4:8]
        si = sems[8:12]
        wid = lax.axis_index("s") * _NC + lax.axis_index("c")
        ebase = wid * _BPW

        def elem0(g):
            return ebase + g * _GE

        def fire_gathers(g, s):
            for e in range(_GE):
                for off, ln in _SPLITS:
                    pltpu.async_copy(
                        table_hbm.at[idx_v.at[s].at[e].at[pl.ds(off, ln)]],
                        rows_v.at[s].at[e].at[pl.ds(off, ln)],
                        sg[s],
                    )

        def drain_gathers(s):
            for e in range(_GE):
                pltpu.make_async_copy(
                    table_hbm.at[pl.ds(0, HIST)], rows_v.at[s].at[e], sg[s]
                ).wait()

        def fire_write(g, s):
            pltpu.async_copy(rows_v.at[s], out_hbm.at[pl.ds(elem0(g), _GE)], sw[s])

        def drain_write(s):
            pltpu.make_async_copy(
                out_hbm.at[pl.ds(0, _GE)], rows_v.at[s], sw[s]
            ).wait()

        def fire_idx(g, s):
            pltpu.async_copy(idx_hbm.at[pl.ds(elem0(g), _GE)], idx_v.at[s], si[s])

        def drain_idx(s):
            pltpu.make_async_copy(
                idx_hbm.at[pl.ds(0, _GE)], idx_v.at[s], si[s]
            ).wait()

        for s in range(3):
            pltpu.sync_copy(idx_hbm.at[pl.ds(elem0(s), _GE)], idx_v.at[s])
        fire_gathers(0, 0)
        fire_gathers(1, 1)

        def body(m, carry):
            g0 = 4 * m

            drain_gathers(0)
            fire_write(g0, 0)

            @pl.when(m >= 1)
            def _():
                drain_write(2)
                drain_idx(2)
                fire_gathers(g0 + 2, 2)

            @pl.when(m < 1)
            def _():
                fire_gathers(2, 2)

            fire_idx(g0 + 3, 3)

            drain_gathers(1)
            fire_write(g0 + 1, 1)

            @pl.when(m >= 1)
            def _():
                drain_write(3)

            drain_idx(3)
            fire_gathers(g0 + 3, 3)

            @pl.when(m < _NI - 1)
            def _():
                fire_idx(g0 + 4, 0)

            drain_gathers(2)
            fire_write(g0 + 2, 2)

            @pl.when(m < _NI - 1)
            def _():
                drain_write(0)
                drain_idx(0)
                fire_gathers(g0 + 4, 0)
                fire_idx(g0 + 5, 1)

            drain_gathers(3)
            fire_write(g0 + 3, 3)

            @pl.when(m < _NI - 1)
            def _():
                drain_write(1)
                drain_idx(1)
                fire_gathers(g0 + 5, 1)
                fire_idx(g0 + 6, 2)

            return carry

        lax.fori_loop(0, _NI, body, 0)

        for s in range(4):
            drain_write(s)

    return k(idx, table)


def kernel(visit_order, pos_embed_weight):
    idx = visit_order.astype(jnp.int32)
    sc_part = _sc_gather(lax.slice_in_dim(idx, 0, _SCB), pos_embed_weight)
    tc_part = jnp.take(pos_embed_weight, lax.slice_in_dim(idx, _SCB, BATCH), axis=0)
    return jnp.concatenate([sc_part, tc_part], axis=0)


# speedup vs baseline: 1.3604x; 1.3604x over previous
import functools

import jax
import jax.numpy as jnp
from jax import lax
from jax.experimental import pallas as pl
from jax.experimental.pallas import tpu as pltpu
from jax.experimental.pallas import tpu_sc as plsc

EMBED_NUM = 1000
EMBED_DIM = 64
BATCH = 16384
HIST = 200

_NC = 2
_NS = 16
_NW = _NC * _NS
_BPW = BATCH // _NW
_GE = 2
_ROWS = BATCH * HIST * EMBED_DIM // 128
_GRW = _GE * HIST * EMBED_DIM // 128
_NG = _BPW // _GE
_NI = _NG // 4
_SPLITS = ((0, 128), (128, 72))


def _sc_gather(idx, table):
    mesh = plsc.VectorSubcoreMesh(core_axis_name="c", subcore_axis_name="s")

    @functools.partial(
        pl.kernel,
        mesh=mesh,
        compiler_params=pltpu.CompilerParams(use_tc_tiling_on_sc=False),
        out_type=jax.ShapeDtypeStruct((BATCH, HIST, EMBED_DIM), jnp.float32),
        scratch_types=[
            pltpu.VMEM((4, _GE, HIST), jnp.int32),
            pltpu.VMEM((4, _GE, HIST, EMBED_DIM), jnp.float32),
            pltpu.SemaphoreType.DMA,
            pltpu.SemaphoreType.DMA,
            pltpu.SemaphoreType.DMA,
            pltpu.SemaphoreType.DMA,
            pltpu.SemaphoreType.DMA,
            pltpu.SemaphoreType.DMA,
            pltpu.SemaphoreType.DMA,
            pltpu.SemaphoreType.DMA,
            pltpu.SemaphoreType.DMA,
            pltpu.SemaphoreType.DMA,
            pltpu.SemaphoreType.DMA,
            pltpu.SemaphoreType.DMA,
        ],
    )
    def k(idx_hbm, table_hbm, out_hbm, idx_v, rows_v, *sems):
        sg = sems[0:4]
        sw = sems[4:8]
        si = sems[8:12]
        wid = lax.axis_index("s") * _NC + lax.axis_index("c")
        ebase = wid * _BPW

        def elem0(g):
            return ebase + g * _GE

        def fire_gathers(g, s):
            for e in range(_GE):
                for off, ln in _SPLITS:
                    pltpu.async_copy(
                        table_hbm.at[idx_v.at[s].at[e].at[pl.ds(off, ln)]],
                        rows_v.at[s].at[e].at[pl.ds(off, ln)],
                        sg[s],
                    )

        def drain_gathers(s):
            for e in range(_GE):
                pltpu.make_async_copy(
                    table_hbm.at[pl.ds(0, HIST)], rows_v.at[s].at[e], sg[s]
                ).wait()

        def fire_write(g, s):
            pltpu.async_copy(rows_v.at[s], out_hbm.at[pl.ds(elem0(g), _GE)], sw[s])

        def drain_write(s):
            pltpu.make_async_copy(
                out_hbm.at[pl.ds(0, _GE)], rows_v.at[s], sw[s]
            ).wait()

        def fire_idx(g, s):
            pltpu.async_copy(idx_hbm.at[pl.ds(elem0(g), _GE)], idx_v.at[s], si[s])

        def drain_idx(s):
            pltpu.make_async_copy(
                idx_hbm.at[pl.ds(0, _GE)], idx_v.at[s], si[s]
            ).wait()

        for s in range(3):
            pltpu.sync_copy(idx_hbm.at[pl.ds(elem0(s), _GE)], idx_v.at[s])
        fire_gathers(0, 0)
        fire_gathers(1, 1)

        def body(m, carry):
            g0 = 4 * m

            drain_gathers(0)
            fire_write(g0, 0)

            @pl.when(m >= 1)
            def _():
                drain_write(2)
                drain_idx(2)
                fire_gathers(g0 + 2, 2)

            @pl.when(m < 1)
            def _():
                fire_gathers(2, 2)

            fire_idx(g0 + 3, 3)

            drain_gathers(1)
            fire_write(g0 + 1, 1)

            @pl.when(m >= 1)
            def _():
                drain_write(3)

            drain_idx(3)
            fire_gathers(g0 + 3, 3)

            @pl.when(m < _NI - 1)
            def _():
                fire_idx(g0 + 4, 0)

            drain_gathers(2)
            fire_write(g0 + 2, 2)

            @pl.when(m < _NI - 1)
            def _():
                drain_write(0)
                drain_idx(0)
                fire_gathers(g0 + 4, 0)
                fire_idx(g0 + 5, 1)

            drain_gathers(3)
            fire_write(g0 + 3, 3)

            @pl.when(m < _NI - 1)
            def _():
                drain_write(1)
                drain_idx(1)
                fire_gathers(g0 + 5, 1)
                fire_idx(g0 + 6, 2)

            return carry

        lax.fori_loop(0, _NI, body, 0)

        for s in range(4):
            drain_write(s)

    return k(idx, table)


def kernel(visit_order, pos_embed_weight):
    return _sc_gather(visit_order.astype(jnp.int32), pos_embed_weight)


# speedup vs baseline: 1.9128x; 1.4060x over previous
import functools

import jax
import jax.numpy as jnp
from jax import lax
from jax.experimental import pallas as pl
from jax.experimental.pallas import tpu as pltpu
from jax.experimental.pallas import tpu_sc as plsc

EMBED_NUM = 1000
EMBED_DIM = 64
BATCH = 16384
HIST = 200

_NC = 2
_NS = 16
_NW = _NC * _NS
_BPW = BATCH // _NW
_GE = 2
_ROWS = BATCH * HIST * EMBED_DIM // 128
_GRW = _GE * HIST * EMBED_DIM // 128
_NG = _BPW // _GE
_NI = _NG // 4
_SPLITS = ((0, 128), (128, 72))


def _sc_gather(idx, table):
    mesh = plsc.VectorSubcoreMesh(core_axis_name="c", subcore_axis_name="s")

    @functools.partial(
        pl.kernel,
        mesh=mesh,
        compiler_params=pltpu.CompilerParams(use_tc_tiling_on_sc=False),
        out_type=jax.ShapeDtypeStruct((BATCH, HIST, EMBED_DIM), jnp.float32),
        scratch_types=[
            pltpu.VMEM((4, _GE, HIST), jnp.int32),
            pltpu.VMEM((4, _GE, HIST, EMBED_DIM), jnp.float32),
            pltpu.VMEM_SHARED((EMBED_NUM, EMBED_DIM), jnp.float32),
            pltpu.SemaphoreType.DMA,
            pltpu.SemaphoreType.DMA,
            pltpu.SemaphoreType.DMA,
            pltpu.SemaphoreType.DMA,
            pltpu.SemaphoreType.DMA,
            pltpu.SemaphoreType.DMA,
            pltpu.SemaphoreType.DMA,
            pltpu.SemaphoreType.DMA,
            pltpu.SemaphoreType.DMA,
            pltpu.SemaphoreType.DMA,
            pltpu.SemaphoreType.DMA,
            pltpu.SemaphoreType.DMA,
        ],
    )
    def k(idx_hbm, table_hbm, out_hbm, idx_v, rows_v, table_sp, *sems):
        sg = sems[0:4]
        sw = sems[4:8]
        si = sems[8:12]
        wid = lax.axis_index("s") * _NC + lax.axis_index("c")
        ebase = wid * _BPW

        @pl.when(lax.axis_index("s") == 0)
        def _():
            pltpu.sync_copy(table_hbm, table_sp)

        plsc.subcore_barrier()

        def elem0(g):
            return ebase + g * _GE

        def fire_gathers(g, s):
            for e in range(_GE):
                for off, ln in _SPLITS:
                    pltpu.async_copy(
                        table_sp.at[idx_v.at[s].at[e].at[pl.ds(off, ln)]],
                        rows_v.at[s].at[e].at[pl.ds(off, ln)],
                        sg[s],
                    )

        def drain_gathers(s):
            for e in range(_GE):
                pltpu.make_async_copy(
                    table_hbm.at[pl.ds(0, HIST)], rows_v.at[s].at[e], sg[s]
                ).wait()

        def fire_write(g, s):
            pltpu.async_copy(rows_v.at[s], out_hbm.at[pl.ds(elem0(g), _GE)], sw[s])

        def drain_write(s):
            pltpu.make_async_copy(
                out_hbm.at[pl.ds(0, _GE)], rows_v.at[s], sw[s]
            ).wait()

        def fire_idx(g, s):
            pltpu.async_copy(idx_hbm.at[pl.ds(elem0(g), _GE)], idx_v.at[s], si[s])

        def drain_idx(s):
            pltpu.make_async_copy(
                idx_hbm.at[pl.ds(0, _GE)], idx_v.at[s], si[s]
            ).wait()

        for s in range(3):
            pltpu.sync_copy(idx_hbm.at[pl.ds(elem0(s), _GE)], idx_v.at[s])
        fire_gathers(0, 0)
        fire_gathers(1, 1)

        def body(m, carry):
            g0 = 4 * m

            drain_gathers(0)
            fire_write(g0, 0)

            @pl.when(m >= 1)
            def _():
                drain_write(2)
                drain_idx(2)
                fire_gathers(g0 + 2, 2)

            @pl.when(m < 1)
            def _():
                fire_gathers(2, 2)

            fire_idx(g0 + 3, 3)

            drain_gathers(1)
            fire_write(g0 + 1, 1)

            @pl.when(m >= 1)
            def _():
                drain_write(3)

            drain_idx(3)
            fire_gathers(g0 + 3, 3)

            @pl.when(m < _NI - 1)
            def _():
                fire_idx(g0 + 4, 0)

            drain_gathers(2)
            fire_write(g0 + 2, 2)

            @pl.when(m < _NI - 1)
            def _():
                drain_write(0)
                drain_idx(0)
                fire_gathers(g0 + 4, 0)
                fire_idx(g0 + 5, 1)

            drain_gathers(3)
            fire_write(g0 + 3, 3)

            @pl.when(m < _NI - 1)
            def _():
                drain_write(1)
                drain_idx(1)
                fire_gathers(g0 + 5, 1)
                fire_idx(g0 + 6, 2)

            return carry

        lax.fori_loop(0, _NI, body, 0)

        for s in range(4):
            drain_write(s)

    return k(idx, table)


def kernel(visit_order, pos_embed_weight):
    return _sc_gather(visit_order.astype(jnp.int32), pos_embed_weight)


# speedup vs baseline: 1.9157x; 1.0015x over previous
import functools

import jax
import jax.numpy as jnp
from jax import lax
from jax.experimental import pallas as pl
from jax.experimental.pallas import tpu as pltpu
from jax.experimental.pallas import tpu_sc as plsc

EMBED_NUM = 1000
EMBED_DIM = 64
BATCH = 16384
HIST = 200

_NC = 2
_NS = 16
_NW = _NC * _NS
_BPW = BATCH // _NW
_GE = 2
_NG = _BPW // _GE
_NI = _NG // 4
_SPLITS = ((0, 128), (128, 72))


def _sc_gather(idx, table):
    mesh = plsc.VectorSubcoreMesh(core_axis_name="c", subcore_axis_name="s")

    @functools.partial(
        pl.kernel,
        mesh=mesh,
        compiler_params=pltpu.CompilerParams(use_tc_tiling_on_sc=False),
        out_type=jax.ShapeDtypeStruct((BATCH, HIST, EMBED_DIM), jnp.float32),
        scratch_types=[
            pltpu.VMEM((4, _GE, HIST), jnp.int32),
            pltpu.VMEM((4, _GE, HIST, EMBED_DIM), jnp.float32),
            pltpu.VMEM_SHARED((EMBED_NUM, EMBED_DIM), jnp.float32),
            pltpu.SemaphoreType.DMA,
            pltpu.SemaphoreType.DMA,
            pltpu.SemaphoreType.DMA,
            pltpu.SemaphoreType.DMA,
            pltpu.SemaphoreType.DMA,
            pltpu.SemaphoreType.DMA,
            pltpu.SemaphoreType.DMA,
            pltpu.SemaphoreType.DMA,
            pltpu.SemaphoreType.DMA,
            pltpu.SemaphoreType.DMA,
            pltpu.SemaphoreType.DMA,
            pltpu.SemaphoreType.DMA,
        ],
    )
    def k(idx_hbm, table_hbm, out_hbm, idx_v, rows_v, table_sp, *sems):
        sg = sems[0:4]
        sw = sems[4:8]
        si = sems[8:12]
        wid = lax.axis_index("s") * _NC + lax.axis_index("c")
        ebase = wid * _BPW

        @pl.when(lax.axis_index("s") == 0)
        def _():
            pltpu.sync_copy(table_hbm, table_sp)

        plsc.subcore_barrier()

        def elem0(g):
            return ebase + g * _GE

        def fire_gathers(g, s):
            for e in range(_GE):
                for off, ln in _SPLITS:
                    pltpu.async_copy(
                        table_sp.at[idx_v.at[s].at[e].at[pl.ds(off, ln)]],
                        rows_v.at[s].at[e].at[pl.ds(off, ln)],
                        sg[s],
                    )

        def drain_gathers(s):
            for e in range(_GE):
                pltpu.make_async_copy(
                    table_hbm.at[pl.ds(0, HIST)], rows_v.at[s].at[e], sg[s]
                ).wait()

        def fire_write(g, s):
            pltpu.async_copy(rows_v.at[s], out_hbm.at[pl.ds(elem0(g), _GE)], sw[s])

        def drain_write(s):
            pltpu.make_async_copy(
                out_hbm.at[pl.ds(0, _GE)], rows_v.at[s], sw[s]
            ).wait()

        def fire_idx(g, s):
            pltpu.async_copy(idx_hbm.at[pl.ds(elem0(g), _GE)], idx_v.at[s], si[s])

        def drain_idx(s):
            pltpu.make_async_copy(
                idx_hbm.at[pl.ds(0, _GE)], idx_v.at[s], si[s]
            ).wait()

        for s in range(3):
            pltpu.sync_copy(idx_hbm.at[pl.ds(elem0(s), _GE)], idx_v.at[s])
        fire_gathers(0, 0)
        fire_gathers(1, 1)

        def body(m, carry):
            g0 = 4 * m

            drain_gathers(0)
            fire_write(g0, 0)

            @pl.when(m >= 1)
            def _():
                drain_write(2)
                drain_idx(2)
                fire_gathers(g0 + 2, 2)

            @pl.when(m < 1)
            def _():
                fire_gathers(2, 2)

            fire_idx(g0 + 3, 3)

            drain_gathers(1)
            fire_write(g0 + 1, 1)

            @pl.when(m >= 1)
            def _():
                drain_write(3)

            drain_idx(3)
            fire_gathers(g0 + 3, 3)

            @pl.when(m < _NI - 1)
            def _():
                fire_idx(g0 + 4, 0)

            drain_gathers(2)
            fire_write(g0 + 2, 2)

            @pl.when(m < _NI - 1)
            def _():
                drain_write(0)
                drain_idx(0)
                fire_gathers(g0 + 4, 0)
                fire_idx(g0 + 5, 1)

            drain_gathers(3)
            fire_write(g0 + 3, 3)

            @pl.when(m < _NI - 1)
            def _():
                drain_write(1)
                drain_idx(1)
                fire_gathers(g0 + 5, 1)
                fire_idx(g0 + 6, 2)

            return carry

        lax.fori_loop(0, _NI, body, 0)

        for s in range(4):
            drain_write(s)

    return k(idx, table)


def kernel(visit_order, pos_embed_weight):
    return _sc_gather(visit_order.astype(jnp.int32), pos_embed_weight)
